# Initial kernel scaffold; baseline (speedup 1.0000x reference)
import jax
import jax.numpy as jnp
from jax.experimental import pallas as pl


def kernel(pred_logits, pred_coords, pred_boxes, orig_target_sizes, input_sizes):
    raise NotImplementedError("write your pallas kernel here")


# speedup vs baseline: 2.8993x; 2.8993x over previous
import jax
import jax.numpy as jnp
from jax.experimental import pallas as pl
from jax.experimental.pallas import tpu as pltpu
import functools

_B, _N, _C, _P, _TOPK = 4, 20000, 80, 32, 300
_NC = _N * _C
_A = 98
_PADNC = _A * 128 * 128
_D = 2 * _P + 4


def _topk_body(lg, cb, ot, lb_o, bo_o, cd_o, sc_o, f3, l1):
    f3[...] = lg[0]
    l1[...] = jnp.max(f3[...], axis=2)

    o0 = ot[0, 0, 0]
    o1 = ot[0, 0, 1]
    cio = jax.lax.broadcasted_iota(jnp.int32, (1, 1, 2 * _P), 2)
    svec = jnp.where(cio % 2 == 0, o0, o1)

    riota = (jax.lax.broadcasted_iota(jnp.int32, (_A, 128), 0) * 128
             + jax.lax.broadcasted_iota(jnp.int32, (_A, 128), 1))
    liota = jax.lax.broadcasted_iota(jnp.int32, (1, 1, 128), 2)
    siota = jax.lax.broadcasted_iota(jnp.int32, (1, 128), 1)
    biota = jax.lax.broadcasted_iota(jnp.int32, (1, 1, 4), 2)
    big = jnp.int32(1 << 30)
    neg_inf = jnp.float32(-jnp.inf)

    def step(i, carry):
        l1v = l1[...]
        m = jnp.max(l1v)
        r = jnp.min(jnp.where(l1v == m, riota, big))
        a = r // 128
        s = r - a * 128
        row = f3[pl.ds(a, 1), pl.ds(s, 1), :]
        lane = jnp.min(jnp.where(row == m, liota, big))
        newrow = jnp.where(liota == lane, neg_inf, row)
        f3[pl.ds(a, 1), pl.ds(s, 1), :] = newrow
        l1row = l1[pl.ds(a, 1), :]
        l1[pl.ds(a, 1), :] = jnp.where(siota == s,
                                       jnp.max(newrow, axis=2), l1row)

        idx = r * 128 + lane
        q = idx // _C
        lab = idx - q * _C

        sc_o[:, pl.ds(i, 1), :] = jnp.reshape(jax.nn.sigmoid(m), (1, 1, 1))
        lb_o[:, pl.ds(i, 1), :] = jnp.reshape(lab, (1, 1, 1))

        row68 = cb[:, pl.ds(q, 1), :]
        cd_o[:, pl.ds(i, 1), :] = row68[:, :, 0:2 * _P] * svec
        cx = row68[0, 0, 2 * _P + 0]
        cy = row68[0, 0, 2 * _P + 1]
        w = row68[0, 0, 2 * _P + 2]
        h = row68[0, 0, 2 * _P + 3]
        x0 = (cx - 0.5 * w) * o0
        y0 = (cy - 0.5 * h) * o1
        x1 = (cx + 0.5 * w) * o0
        y1 = (cy + 0.5 * h) * o1
        bo_o[:, pl.ds(i, 1), :] = jnp.where(
            biota == 0, x0, jnp.where(biota == 1, y0,
                                      jnp.where(biota == 2, x1, y1)))
        return carry

    jax.lax.fori_loop(0, _TOPK, step, 0)


@functools.partial(jax.jit, static_argnames=("interpret",))
def _run(pred_logits, pred_coords, pred_boxes, orig_target_sizes, interpret=False):
    flat = pred_logits.reshape(_B, _NC)
    flatp = jnp.pad(flat, ((0, 0), (0, _PADNC - _NC)),
                    constant_values=-jnp.inf)
    f3 = flatp.reshape(_B, _A, 128, 128)
    comb = jnp.concatenate(
        [pred_coords.reshape(_B, _N, 2 * _P), pred_boxes], axis=2)
    ots = orig_target_sizes.reshape(_B, 1, 2)

    grid = (_B,)
    lb, bo, cd, sc = pl.pallas_call(
        _topk_body,
        grid=grid,
        in_specs=[
            pl.BlockSpec((1, _A, 128, 128), lambda b: (b, 0, 0, 0)),
            pl.BlockSpec((1, _N, _D), lambda b: (b, 0, 0)),
            pl.BlockSpec((1, 1, 2), lambda b: (b, 0, 0)),
        ],
        out_specs=[
            pl.BlockSpec((1, _TOPK, 1), lambda b: (b, 0, 0)),
            pl.BlockSpec((1, _TOPK, 4), lambda b: (b, 0, 0)),
            pl.BlockSpec((1, _TOPK, 2 * _P), lambda b: (b, 0, 0)),
            pl.BlockSpec((1, _TOPK, 1), lambda b: (b, 0, 0)),
        ],
        out_shape=[
            jax.ShapeDtypeStruct((_B, _TOPK, 1), jnp.int32),
            jax.ShapeDtypeStruct((_B, _TOPK, 4), jnp.float32),
            jax.ShapeDtypeStruct((_B, _TOPK, 2 * _P), jnp.float32),
            jax.ShapeDtypeStruct((_B, _TOPK, 1), jnp.float32),
        ],
        scratch_shapes=[
            pltpu.VMEM((_A, 128, 128), jnp.float32),
            pltpu.VMEM((_A, 128), jnp.float32),
        ],
        interpret=interpret,
    )(f3, comb, ots)

    labels = lb.reshape(_B, _TOPK)
    boxes_sel = bo
    coords_sel = cd.reshape(_B, _TOPK, _P, 2)
    top_scores = sc.reshape(_B, _TOPK)
    return labels, boxes_sel, coords_sel, top_scores


def kernel(pred_logits, pred_coords, pred_boxes, orig_target_sizes, input_sizes):
    return _run(pred_logits, pred_coords, pred_boxes, orig_target_sizes)


# speedup vs baseline: 5.3274x; 1.8375x over previous
import jax
import jax.numpy as jnp
from jax.experimental import pallas as pl
from jax.experimental.pallas import tpu as pltpu
import functools

_B, _N, _C, _P, _TOPK = 4, 20000, 80, 32, 300
_D = 2 * _P + 4
_QB = 2048
_NBLK = 10
_T = 160


def _body(lg, cb, ot, lb_o, bo_o, cd_o, sc_o, f3, l1):
    a = pl.program_id(1)

    x = lg[...]
    qabs = (jax.lax.broadcasted_iota(jnp.int32, (1, _QB, _C), 1)
            + a * _QB)
    x = jnp.where(qabs < _N, x, -jnp.inf)
    for j in range(_QB // 128):
        xj = x[:, j * 128:(j + 1) * 128, :]
        t = a * (_QB // 128) + j
        f3[pl.ds(t, 1), :, :] = xj
        l1[pl.ds(t, 1), :] = jnp.max(xj, axis=2)

    @pl.when(a == _NBLK - 1)
    def _select():
        o0 = ot[0, 0, 0]
        o1 = ot[0, 0, 1]
        cio = jax.lax.broadcasted_iota(jnp.int32, (1, 1, 2 * _P), 2)
        svec = jnp.where(cio % 2 == 0, o0, o1)

        riota = (jax.lax.broadcasted_iota(jnp.int32, (_T, 128), 0) * 128
                 + jax.lax.broadcasted_iota(jnp.int32, (_T, 128), 1))
        liota = jax.lax.broadcasted_iota(jnp.int32, (1, 1, _C), 2)
        siota = jax.lax.broadcasted_iota(jnp.int32, (1, 128), 1)
        biota = jax.lax.broadcasted_iota(jnp.int32, (1, 1, 4), 2)
        big = jnp.int32(1 << 30)
        neg_inf = jnp.float32(-jnp.inf)

        def step(i, carry):
            l1v = l1[...]
            m = jnp.max(l1v)
            q = jnp.min(jnp.where(l1v == m, riota, big))
            t = q // 128
            s = q - t * 128
            row = f3[pl.ds(t, 1), pl.ds(s, 1), :]
            lane = jnp.min(jnp.where(row == m, liota, big))
            newrow = jnp.where(liota == lane, neg_inf, row)
            f3[pl.ds(t, 1), pl.ds(s, 1), :] = newrow
            l1row = l1[pl.ds(t, 1), :]
            l1[pl.ds(t, 1), :] = jnp.where(siota == s,
                                           jnp.max(newrow, axis=2), l1row)

            sc_o[:, pl.ds(i, 1), :] = jnp.reshape(jax.nn.sigmoid(m),
                                                  (1, 1, 1))
            lb_o[:, pl.ds(i, 1), :] = jnp.reshape(lane, (1, 1, 1))

            row68 = cb[:, pl.ds(q, 1), :]
            cd_o[:, pl.ds(i, 1), :] = row68[:, :, 0:2 * _P] * svec
            cx = row68[0, 0, 2 * _P + 0]
            cy = row68[0, 0, 2 * _P + 1]
            w = row68[0, 0, 2 * _P + 2]
            h = row68[0, 0, 2 * _P + 3]
            x0 = (cx - 0.5 * w) * o0
            y0 = (cy - 0.5 * h) * o1
            x1 = (cx + 0.5 * w) * o0
            y1 = (cy + 0.5 * h) * o1
            bo_o[:, pl.ds(i, 1), :] = jnp.where(
                biota == 0, x0, jnp.where(biota == 1, y0,
                                          jnp.where(biota == 2, x1, y1)))
            return carry

        jax.lax.fori_loop(0, _TOPK, step, 0)


@functools.partial(jax.jit, static_argnames=("interpret",))
def _run(pred_logits, pred_coords, pred_boxes, orig_target_sizes, interpret=False):
    comb = jnp.concatenate(
        [pred_coords.reshape(_B, _N, 2 * _P), pred_boxes], axis=2)
    ots = orig_target_sizes.reshape(_B, 1, 2)

    grid = (_B, _NBLK)
    lb, bo, cd, sc = pl.pallas_call(
        _body,
        grid=grid,
        in_specs=[
            pl.BlockSpec((1, _QB, _C), lambda b, a: (b, a, 0)),
            pl.BlockSpec((1, _N, _D), lambda b, a: (b, 0, 0)),
            pl.BlockSpec((1, 1, 2), lambda b, a: (b, 0, 0)),
        ],
        out_specs=[
            pl.BlockSpec((1, _TOPK, 1), lambda b, a: (b, 0, 0)),
            pl.BlockSpec((1, _TOPK, 4), lambda b, a: (b, 0, 0)),
            pl.BlockSpec((1, _TOPK, 2 * _P), lambda b, a: (b, 0, 0)),
            pl.BlockSpec((1, _TOPK, 1), lambda b, a: (b, 0, 0)),
        ],
        out_shape=[
            jax.ShapeDtypeStruct((_B, _TOPK, 1), jnp.int32),
            jax.ShapeDtypeStruct((_B, _TOPK, 4), jnp.float32),
            jax.ShapeDtypeStruct((_B, _TOPK, 2 * _P), jnp.float32),
            jax.ShapeDtypeStruct((_B, _TOPK, 1), jnp.float32),
        ],
        scratch_shapes=[
            pltpu.VMEM((_T, 128, _C), jnp.float32),
            pltpu.VMEM((_T, 128), jnp.float32),
        ],
        interpret=interpret,
    )(pred_logits, comb, ots)

    labels = lb.reshape(_B, _TOPK)
    boxes_sel = bo
    coords_sel = cd.reshape(_B, _TOPK, _P, 2)
    top_scores = sc.reshape(_B, _TOPK)
    return labels, boxes_sel, coords_sel, top_scores


def kernel(pred_logits, pred_coords, pred_boxes, orig_target_sizes, input_sizes):
    return _run(pred_logits, pred_coords, pred_boxes, orig_target_sizes)
